# Initial kernel scaffold; baseline (speedup 1.0000x reference)
import jax
import jax.numpy as jnp
from jax.experimental import pallas as pl


def kernel(prediction, target):
    raise NotImplementedError("write your pallas kernel here")


# speedup vs baseline: 30.0443x; 30.0443x over previous
import functools

import jax
import jax.numpy as jnp
from jax import lax
from jax.experimental import pallas as pl
from jax.experimental.pallas import tpu as pltpu, tpu_sc as plsc

BINS = 100
NC = 2
NS = 16
NW = NC * NS
L = 16

N = 32 * 3 * 512 * 512
PER_TILE = N // NW
CHUNK = 32768
NCHUNK = PER_TILE // CHUNK

_mesh = plsc.VectorSubcoreMesh(core_axis_name="c", subcore_axis_name="s")


def _wid():
    return lax.axis_index("s") * NC + lax.axis_index("c")


@functools.partial(
    pl.kernel,
    out_type=jax.ShapeDtypeStruct((NW, 2, L), jnp.float32),
    mesh=_mesh,
    compiler_params=pltpu.CompilerParams(needs_layout_passes=False),
    scratch_types=[
        pltpu.VMEM((CHUNK,), jnp.float32),
        pltpu.VMEM((2, L), jnp.float32),
    ],
)
def _minmax_kernel(p_hbm, t_hbm, out_hbm, buf, mm):
    base = _wid() * PER_TILE

    def scan_array(arr_ref, carry):
        def chunk_body(c, carry):
            pltpu.sync_copy(arr_ref.at[pl.ds(base + c * CHUNK, CHUNK)], buf)

            def vec_body(i, carry):
                vmin, vmax = carry
                x = buf[pl.ds(i * L, L)]
                return jnp.minimum(vmin, x), jnp.maximum(vmax, x)

            return lax.fori_loop(0, CHUNK // L, vec_body, carry)

        return lax.fori_loop(0, NCHUNK, chunk_body, carry)

    init = (jnp.full((L,), jnp.inf, jnp.float32),
            jnp.full((L,), -jnp.inf, jnp.float32))
    vmin, vmax = scan_array(t_hbm, scan_array(p_hbm, init))
    mm[0, :] = vmin
    mm[1, :] = vmax
    pltpu.sync_copy(mm, out_hbm.at[_wid()])


@functools.partial(
    pl.kernel,
    out_type=jax.ShapeDtypeStruct((NW, 2, BINS, L), jnp.float32),
    mesh=_mesh,
    compiler_params=pltpu.CompilerParams(needs_layout_passes=False),
    scratch_types=[
        pltpu.VMEM((CHUNK,), jnp.float32),
        pltpu.VMEM((BINS, L), jnp.float32),
        pltpu.VMEM((BINS, L), jnp.float32),
        pltpu.VMEM((L,), jnp.float32),
        pltpu.VMEM((L,), jnp.float32),
        pltpu.VMEM((L,), jnp.float32),
    ],
)
def _hist_kernel(p_hbm, t_hbm, lo_hbm, w_hbm, hi_hbm, out_hbm,
                 buf, hp, ht, lov, wv, hiv):
    base = _wid() * PER_TILE
    pltpu.sync_copy(lo_hbm, lov)
    pltpu.sync_copy(w_hbm, wv)
    pltpu.sync_copy(hi_hbm, hiv)
    lo = lov[...]
    w = wv[...]
    hi = hiv[...]
    lane = lax.iota(jnp.int32, L)
    ones = jnp.full((L,), 1.0, jnp.float32)
    zeros = jnp.zeros((L,), jnp.float32)

    def zero_hist(h):
        def b(i, _):
            h[i, :] = zeros
            return 0
        lax.fori_loop(0, BINS, b, 0)

    def hist_array(arr_ref, h):
        def chunk_body(c, _):
            pltpu.sync_copy(arr_ref.at[pl.ds(base + c * CHUNK, CHUNK)], buf)

            def vec_body(i, _):
                x = buf[pl.ds(i * L, L)]
                scaled = (x - lo) / w * jnp.float32(BINS)
                idx = scaled.astype(jnp.int32)
                idx = jnp.minimum(jnp.maximum(idx, 0), BINS - 1)
                valid = (x >= lo) & (x <= hi)
                plsc.addupdate_scatter(h, [idx, lane], ones, mask=valid)
                return 0

            lax.fori_loop(0, CHUNK // L, vec_body, 0)
            return 0

        lax.fori_loop(0, NCHUNK, chunk_body, 0)

    zero_hist(hp)
    zero_hist(ht)
    hist_array(p_hbm, hp)
    hist_array(t_hbm, ht)
    pltpu.sync_copy(hp, out_hbm.at[_wid(), 0])
    pltpu.sync_copy(ht, out_hbm.at[_wid(), 1])


def kernel(prediction, target):
    p = prediction.reshape(-1)
    t = target.reshape(-1)
    mm = _minmax_kernel(p, t)
    minv = jnp.min(mm[:, 0, :])
    maxv = jnp.max(mm[:, 1, :])
    lo = minv + 0.1
    width = maxv - lo
    lo16 = jnp.full((L,), lo, jnp.float32)
    w16 = jnp.full((L,), width, jnp.float32)
    hi16 = jnp.full((L,), maxv, jnp.float32)
    parts = _hist_kernel(p, t, lo16, w16, hi16)
    h = parts.sum(axis=(0, 3))
    return jnp.mean(jnp.abs(h[0] - h[1]))


# speedup vs baseline: 36.2444x; 1.2064x over previous
import functools

import jax
import jax.numpy as jnp
from jax import lax
from jax.experimental import pallas as pl
from jax.experimental.pallas import tpu as pltpu, tpu_sc as plsc

BINS = 100
NC = 2
NS = 16
NW = NC * NS
L = 16

N = 32 * 3 * 512 * 512
PER_TILE = N // NW
CHUNK = 32768
NCHUNK = PER_TILE // CHUNK

_mesh = plsc.VectorSubcoreMesh(core_axis_name="c", subcore_axis_name="s")


def _wid():
    return lax.axis_index("s") * NC + lax.axis_index("c")


@functools.partial(
    pl.kernel,
    out_type=jax.ShapeDtypeStruct((NW, 2, L), jnp.float32),
    mesh=_mesh,
    compiler_params=pltpu.CompilerParams(needs_layout_passes=False),
    scratch_types=[
        pltpu.VMEM((CHUNK,), jnp.float32),
        pltpu.VMEM((2, L), jnp.float32),
    ],
)
def _minmax_kernel(p_hbm, t_hbm, out_hbm, buf, mm):
    base = _wid() * PER_TILE

    U = 8

    def scan_array(arr_ref, carry):
        def chunk_body(c, carry):
            pltpu.sync_copy(arr_ref.at[pl.ds(base + c * CHUNK, CHUNK)], buf)

            def vec_body(i, carry):
                vmin, vmax = carry
                for u in range(U):
                    x = buf[pl.ds((i * U + u) * L, L)]
                    vmin = jnp.minimum(vmin, x)
                    vmax = jnp.maximum(vmax, x)
                return vmin, vmax

            return lax.fori_loop(0, CHUNK // L // U, vec_body, carry)

        return lax.fori_loop(0, NCHUNK, chunk_body, carry)

    init = (jnp.full((L,), jnp.inf, jnp.float32),
            jnp.full((L,), -jnp.inf, jnp.float32))
    vmin, vmax = scan_array(t_hbm, scan_array(p_hbm, init))
    mm[0, :] = vmin
    mm[1, :] = vmax
    pltpu.sync_copy(mm, out_hbm.at[_wid()])


@functools.partial(
    pl.kernel,
    out_type=jax.ShapeDtypeStruct((NW, 2, BINS, L), jnp.float32),
    mesh=_mesh,
    compiler_params=pltpu.CompilerParams(needs_layout_passes=False),
    scratch_types=[
        pltpu.VMEM((CHUNK,), jnp.float32),
        pltpu.VMEM((BINS, L), jnp.float32),
        pltpu.VMEM((BINS, L), jnp.float32),
        pltpu.VMEM((L,), jnp.float32),
        pltpu.VMEM((L,), jnp.float32),
        pltpu.VMEM((L,), jnp.float32),
    ],
)
def _hist_kernel(p_hbm, t_hbm, lo_hbm, w_hbm, hi_hbm, out_hbm,
                 buf, hp, ht, lov, wv, hiv):
    base = _wid() * PER_TILE
    pltpu.sync_copy(lo_hbm, lov)
    pltpu.sync_copy(w_hbm, wv)
    pltpu.sync_copy(hi_hbm, hiv)
    lo = lov[...]
    w = wv[...]
    hi = hiv[...]
    lane = lax.iota(jnp.int32, L)
    ones = jnp.full((L,), 1.0, jnp.float32)
    zeros = jnp.zeros((L,), jnp.float32)

    def zero_hist(h):
        def b(i, _):
            h[i, :] = zeros
            return 0
        lax.fori_loop(0, BINS, b, 0)

    U = 8

    def hist_array(arr_ref, h):
        def chunk_body(c, _):
            pltpu.sync_copy(arr_ref.at[pl.ds(base + c * CHUNK, CHUNK)], buf)

            def vec_body(i, _):
                for u in range(U):
                    x = buf[pl.ds((i * U + u) * L, L)]
                    scaled = (x - lo) / w * jnp.float32(BINS)
                    idx = scaled.astype(jnp.int32)
                    idx = jnp.minimum(jnp.maximum(idx, 0), BINS - 1)
                    valid = (x >= lo) & (x <= hi)
                    plsc.addupdate_scatter(h, [idx, lane], ones, mask=valid)
                return 0

            lax.fori_loop(0, CHUNK // L // U, vec_body, 0)
            return 0

        lax.fori_loop(0, NCHUNK, chunk_body, 0)

    zero_hist(hp)
    zero_hist(ht)
    hist_array(p_hbm, hp)
    hist_array(t_hbm, ht)
    pltpu.sync_copy(hp, out_hbm.at[_wid(), 0])
    pltpu.sync_copy(ht, out_hbm.at[_wid(), 1])


def kernel(prediction, target):
    p = prediction.reshape(-1)
    t = target.reshape(-1)
    mm = _minmax_kernel(p, t)
    minv = jnp.min(mm[:, 0, :])
    maxv = jnp.max(mm[:, 1, :])
    lo = minv + 0.1
    width = maxv - lo
    lo16 = jnp.full((L,), lo, jnp.float32)
    w16 = jnp.full((L,), width, jnp.float32)
    hi16 = jnp.full((L,), maxv, jnp.float32)
    parts = _hist_kernel(p, t, lo16, w16, hi16)
    h = parts.sum(axis=(0, 3))
    return jnp.mean(jnp.abs(h[0] - h[1]))


# speedup vs baseline: 131.5874x; 3.6306x over previous
import functools

import jax
import jax.numpy as jnp
from jax import lax
from jax.experimental import pallas as pl
from jax.experimental.pallas import tpu as pltpu, tpu_sc as plsc

BINS = 100
NC = 2
NS = 16
NW = NC * NS
L = 16

N = 32 * 3 * 512 * 512
PER_TILE = N // NW
CHUNK = 32768
NCHUNK = PER_TILE // CHUNK

_mesh = plsc.VectorSubcoreMesh(core_axis_name="c", subcore_axis_name="s")


def _wid():
    return lax.axis_index("s") * NC + lax.axis_index("c")


def _stream_chunks(arr_ref, base, bufs, sems, consume, carry):
    for b in range(2):
        pltpu.async_copy(
            arr_ref.at[pl.ds(base + b * CHUNK, CHUNK)], bufs[b], sems[b])

    def pair_body(g, carry):
        for b in range(2):
            c = 2 * g + b
            pltpu.make_async_copy(
                arr_ref.at[pl.ds(0, CHUNK)], bufs[b], sems[b]).wait()
            carry = consume(bufs[b], carry)

            @pl.when(c + 2 < NCHUNK)
            def _():
                pltpu.async_copy(
                    arr_ref.at[pl.ds(base + (c + 2) * CHUNK, CHUNK)],
                    bufs[b], sems[b])
        return carry

    return lax.fori_loop(0, NCHUNK // 2, pair_body, carry)


@functools.partial(
    pl.kernel,
    out_type=jax.ShapeDtypeStruct((NW, 2, L), jnp.float32),
    mesh=_mesh,
    compiler_params=pltpu.CompilerParams(needs_layout_passes=False),
    scratch_types=[
        pltpu.VMEM((CHUNK,), jnp.float32),
        pltpu.VMEM((CHUNK,), jnp.float32),
        pltpu.VMEM((2, L), jnp.float32),
        pltpu.SemaphoreType.DMA,
        pltpu.SemaphoreType.DMA,
    ],
)
def _minmax_kernel(p_hbm, t_hbm, out_hbm, buf0, buf1, mm, sem0, sem1):
    base = _wid() * PER_TILE

    def consume(buf, carry):
        @plsc.parallel_loop(0, CHUNK // L, unroll=8, carry=carry)
        def vec_body(i, carry):
            vmin, vmax = carry
            x = buf[pl.ds(i * L, L)]
            return jnp.minimum(vmin, x), jnp.maximum(vmax, x)

        return vec_body

    def scan_array(arr_ref, carry):
        return _stream_chunks(
            arr_ref, base, (buf0, buf1), (sem0, sem1), consume, carry)

    init = (jnp.full((L,), jnp.inf, jnp.float32),
            jnp.full((L,), -jnp.inf, jnp.float32))
    vmin, vmax = scan_array(t_hbm, scan_array(p_hbm, init))
    mm[0, :] = vmin
    mm[1, :] = vmax
    pltpu.sync_copy(mm, out_hbm.at[_wid()])


@functools.partial(
    pl.kernel,
    out_type=jax.ShapeDtypeStruct((NW, 2, BINS, L), jnp.float32),
    mesh=_mesh,
    compiler_params=pltpu.CompilerParams(needs_layout_passes=False),
    scratch_types=[
        pltpu.VMEM((CHUNK,), jnp.float32),
        pltpu.VMEM((CHUNK,), jnp.float32),
        pltpu.VMEM((BINS, L), jnp.float32),
        pltpu.VMEM((BINS, L), jnp.float32),
        pltpu.VMEM((L,), jnp.float32),
        pltpu.VMEM((L,), jnp.float32),
        pltpu.VMEM((L,), jnp.float32),
        pltpu.SemaphoreType.DMA,
        pltpu.SemaphoreType.DMA,
    ],
)
def _hist_kernel(p_hbm, t_hbm, lo_hbm, w_hbm, hi_hbm, out_hbm,
                 buf0, buf1, hp, ht, lov, wv, hiv, sem0, sem1):
    base = _wid() * PER_TILE
    pltpu.sync_copy(lo_hbm, lov)
    pltpu.sync_copy(w_hbm, wv)
    pltpu.sync_copy(hi_hbm, hiv)
    lo = lov[...]
    w = wv[...]
    hi = hiv[...]
    lane = lax.iota(jnp.int32, L)
    ones = jnp.full((L,), 1.0, jnp.float32)
    zeros = jnp.zeros((L,), jnp.float32)

    def zero_hist(h):
        def b(i, _):
            h[i, :] = zeros
            return 0
        lax.fori_loop(0, BINS, b, 0)

    def hist_array(arr_ref, h):
        def consume(buf, carry):
            @plsc.parallel_loop(0, CHUNK // L, unroll=8)
            def vec_body(i):
                x = buf[pl.ds(i * L, L)]
                scaled = (x - lo) / w * jnp.float32(BINS)
                idx = scaled.astype(jnp.int32)
                idx = jnp.minimum(jnp.maximum(idx, 0), BINS - 1)
                valid = (x >= lo) & (x <= hi)
                plsc.addupdate_scatter(h, [idx, lane], ones, mask=valid)

            return carry

        _stream_chunks(arr_ref, base, (buf0, buf1), (sem0, sem1), consume, 0)

    zero_hist(hp)
    zero_hist(ht)
    hist_array(p_hbm, hp)
    hist_array(t_hbm, ht)
    pltpu.sync_copy(hp, out_hbm.at[_wid(), 0])
    pltpu.sync_copy(ht, out_hbm.at[_wid(), 1])


def kernel(prediction, target):
    p = prediction.reshape(-1)
    t = target.reshape(-1)
    mm = _minmax_kernel(p, t)
    minv = jnp.min(mm[:, 0, :])
    maxv = jnp.max(mm[:, 1, :])
    lo = minv + 0.1
    width = maxv - lo
    lo16 = jnp.full((L,), lo, jnp.float32)
    w16 = jnp.full((L,), width, jnp.float32)
    hi16 = jnp.full((L,), maxv, jnp.float32)
    parts = _hist_kernel(p, t, lo16, w16, hi16)
    h = parts.sum(axis=(0, 3))
    return jnp.mean(jnp.abs(h[0] - h[1]))


# speedup vs baseline: 204.5921x; 1.5548x over previous
import functools

import jax
import jax.numpy as jnp
from jax import lax
from jax.experimental import pallas as pl
from jax.experimental.pallas import tpu as pltpu, tpu_sc as plsc

BINS = 100
NC = 2
NS = 16
NW = NC * NS
L = 16

N = 32 * 3 * 512 * 512
COLS = 512
ROWS = N // COLS
ROWS_PER_TILE = ROWS // NW
CROWS = 64
CHUNK = CROWS * COLS
NCHUNK = ROWS_PER_TILE // CROWS
NVEC = CHUNK // L

_mesh = plsc.VectorSubcoreMesh(core_axis_name="c", subcore_axis_name="s")
_params = pltpu.CompilerParams(needs_layout_passes=False,
                               use_tc_tiling_on_sc=True)


def _wid():
    return lax.axis_index("s") * NC + lax.axis_index("c")


def _stream_chunks(arr_ref, rbase, bufs, sems, consume, carry):
    for b in range(2):
        pltpu.async_copy(
            arr_ref.at[pl.ds(rbase + b * CROWS, CROWS), :], bufs[b], sems[b])

    def pair_body(g, carry):
        for b in range(2):
            c = 2 * g + b
            pltpu.make_async_copy(
                arr_ref.at[pl.ds(0, CROWS), :], bufs[b], sems[b]).wait()
            carry = consume(bufs[b], carry)

            @pl.when(c + 2 < NCHUNK)
            def _():
                pltpu.async_copy(
                    arr_ref.at[pl.ds(rbase + (c + 2) * CROWS, CROWS), :],
                    bufs[b], sems[b])
        return carry

    return lax.fori_loop(0, NCHUNK // 2, pair_body, carry)


@functools.partial(
    pl.kernel,
    out_type=jax.ShapeDtypeStruct((NW, 2, L), jnp.float32),
    mesh=_mesh,
    compiler_params=_params,
    scratch_types=[
        pltpu.VMEM((CROWS, COLS), jnp.float32),
        pltpu.VMEM((CROWS, COLS), jnp.float32),
        pltpu.VMEM((2, L), jnp.float32),
        pltpu.SemaphoreType.DMA,
        pltpu.SemaphoreType.DMA,
    ],
)
def _minmax_kernel(p_hbm, t_hbm, out_hbm, buf0, buf1, mm, sem0, sem1):
    rbase = _wid() * ROWS_PER_TILE

    def consume(buf, carry):
        @plsc.parallel_loop(0, NVEC, unroll=8, carry=carry)
        def vec_body(i, carry):
            vmin, vmax = carry
            x = buf[i // (COLS // L), pl.ds((i % (COLS // L)) * L, L)]
            return jnp.minimum(vmin, x), jnp.maximum(vmax, x)

        return vec_body

    def scan_array(arr_ref, carry):
        return _stream_chunks(
            arr_ref, rbase, (buf0, buf1), (sem0, sem1), consume, carry)

    init = (jnp.full((L,), jnp.inf, jnp.float32),
            jnp.full((L,), -jnp.inf, jnp.float32))
    vmin, vmax = scan_array(t_hbm, scan_array(p_hbm, init))
    mm[0, :] = vmin
    mm[1, :] = vmax
    pltpu.sync_copy(mm, out_hbm.at[_wid()])


@functools.partial(
    pl.kernel,
    out_type=jax.ShapeDtypeStruct((NW, 2, BINS, L), jnp.float32),
    mesh=_mesh,
    compiler_params=_params,
    scratch_types=[
        pltpu.VMEM((CROWS, COLS), jnp.float32),
        pltpu.VMEM((CROWS, COLS), jnp.float32),
        pltpu.VMEM((BINS, L), jnp.float32),
        pltpu.VMEM((BINS, L), jnp.float32),
        pltpu.VMEM((L,), jnp.float32),
        pltpu.VMEM((L,), jnp.float32),
        pltpu.SemaphoreType.DMA,
        pltpu.SemaphoreType.DMA,
    ],
)
def _hist_kernel(p_hbm, t_hbm, lo_hbm, w_hbm, out_hbm,
                 buf0, buf1, hp, ht, lov, wv, sem0, sem1):
    rbase = _wid() * ROWS_PER_TILE
    pltpu.sync_copy(lo_hbm, lov)
    pltpu.sync_copy(w_hbm, wv)
    lo = lov[...]
    w = wv[...]
    lane = lax.iota(jnp.int32, L)
    ones = jnp.full((L,), 1.0, jnp.float32)
    zeros = jnp.zeros((L,), jnp.float32)

    def zero_hist(h):
        def b(i, _):
            h[i, :] = zeros
            return 0
        lax.fori_loop(0, BINS, b, 0)

    def hist_array(arr_ref, h):
        def consume(buf, carry):
            @plsc.parallel_loop(0, NVEC, unroll=8)
            def vec_body(i):
                x = buf[i // (COLS // L), pl.ds((i % (COLS // L)) * L, L)]
                scaled = (x - lo) / w * jnp.float32(BINS)
                idx = scaled.astype(jnp.int32)
                idxu = plsc.bitcast(idx, jnp.uint32)
                valid = (x >= lo) & (idxu <= jnp.uint32(BINS))
                slot = plsc.bitcast(
                    jnp.minimum(idxu, jnp.uint32(BINS - 1)), jnp.int32)
                plsc.addupdate_scatter(h, [slot, lane], ones, mask=valid)

            return carry

        _stream_chunks(arr_ref, rbase, (buf0, buf1), (sem0, sem1), consume, 0)

    zero_hist(hp)
    zero_hist(ht)
    hist_array(p_hbm, hp)
    hist_array(t_hbm, ht)
    pltpu.sync_copy(hp, out_hbm.at[_wid(), 0])
    pltpu.sync_copy(ht, out_hbm.at[_wid(), 1])


def kernel(prediction, target):
    p = prediction.reshape(ROWS, COLS)
    t = target.reshape(ROWS, COLS)
    mm = _minmax_kernel(p, t)
    minv = jnp.min(mm[:, 0, :])
    maxv = jnp.max(mm[:, 1, :])
    lo = minv + 0.1
    width = maxv - lo
    lo16 = jnp.full((L,), lo, jnp.float32)
    w16 = jnp.full((L,), width, jnp.float32)
    parts = _hist_kernel(p, t, lo16, w16)
    h = parts.sum(axis=(0, 3))
    return jnp.mean(jnp.abs(h[0] - h[1]))


# speedup vs baseline: 217.6190x; 1.0637x over previous
import functools

import jax
import jax.numpy as jnp
from jax import lax
from jax.experimental import pallas as pl
from jax.experimental.pallas import tpu as pltpu, tpu_sc as plsc

BINS = 100
NC = 2
NS = 16
NW = NC * NS
L = 16

N = 32 * 3 * 512 * 512
COLS = 512
ROWS = N // COLS
ROWS_PER_TILE = ROWS // NW
CROWS = 64
CHUNK = CROWS * COLS
NCHUNK = ROWS_PER_TILE // CROWS
NVEC = CHUNK // L

_mesh = plsc.VectorSubcoreMesh(core_axis_name="c", subcore_axis_name="s")
_params = pltpu.CompilerParams(needs_layout_passes=False,
                               use_tc_tiling_on_sc=True)


def _wid():
    return lax.axis_index("s") * NC + lax.axis_index("c")


def _stream_chunks(arr_ref, rbase, bufs, sems, consume, carry):
    for b in range(2):
        pltpu.async_copy(
            arr_ref.at[pl.ds(rbase + b * CROWS, CROWS), :], bufs[b], sems[b])

    def pair_body(g, carry):
        for b in range(2):
            c = 2 * g + b
            pltpu.make_async_copy(
                arr_ref.at[pl.ds(0, CROWS), :], bufs[b], sems[b]).wait()
            carry = consume(bufs[b], carry)

            @pl.when(c + 2 < NCHUNK)
            def _():
                pltpu.async_copy(
                    arr_ref.at[pl.ds(rbase + (c + 2) * CROWS, CROWS), :],
                    bufs[b], sems[b])
        return carry

    return lax.fori_loop(0, NCHUNK // 2, pair_body, carry)


@functools.partial(
    pl.kernel,
    out_type=jax.ShapeDtypeStruct((NW, 2, L), jnp.float32),
    mesh=_mesh,
    compiler_params=_params,
    scratch_types=[
        pltpu.VMEM((CROWS, COLS), jnp.float32),
        pltpu.VMEM((CROWS, COLS), jnp.float32),
        pltpu.VMEM((2, L), jnp.float32),
        pltpu.SemaphoreType.DMA,
        pltpu.SemaphoreType.DMA,
    ],
)
def _minmax_kernel(p_hbm, t_hbm, out_hbm, buf0, buf1, mm, sem0, sem1):
    rbase = _wid() * ROWS_PER_TILE

    def consume(buf, carry):
        @plsc.parallel_loop(0, NVEC, unroll=8, carry=carry)
        def vec_body(i, carry):
            vmin, vmax = carry
            x = buf[i // (COLS // L), pl.ds((i % (COLS // L)) * L, L)]
            return jnp.minimum(vmin, x), jnp.maximum(vmax, x)

        return vec_body

    def scan_array(arr_ref, carry):
        return _stream_chunks(
            arr_ref, rbase, (buf0, buf1), (sem0, sem1), consume, carry)

    init = (jnp.full((L,), jnp.inf, jnp.float32),
            jnp.full((L,), -jnp.inf, jnp.float32))
    vmin, vmax = scan_array(t_hbm, scan_array(p_hbm, init))
    mm[0, :] = vmin
    mm[1, :] = vmax
    pltpu.sync_copy(mm, out_hbm.at[_wid()])


@functools.partial(
    pl.kernel,
    out_type=jax.ShapeDtypeStruct((NW, 2, BINS + 1, L), jnp.float32),
    mesh=_mesh,
    compiler_params=_params,
    scratch_types=[
        pltpu.VMEM((CROWS, COLS), jnp.float32),
        pltpu.VMEM((CROWS, COLS), jnp.float32),
        pltpu.VMEM((BINS + 1, L), jnp.float32),
        pltpu.VMEM((BINS + 1, L), jnp.float32),
        pltpu.VMEM((L,), jnp.float32),
        pltpu.VMEM((L,), jnp.float32),
        pltpu.SemaphoreType.DMA,
        pltpu.SemaphoreType.DMA,
    ],
)
def _hist_kernel(p_hbm, t_hbm, lo_hbm, w_hbm, out_hbm,
                 buf0, buf1, hp, ht, lov, wv, sem0, sem1):
    rbase = _wid() * ROWS_PER_TILE
    pltpu.sync_copy(lo_hbm, lov)
    pltpu.sync_copy(w_hbm, wv)
    lo = lov[...]
    w = wv[...]
    lane = lax.iota(jnp.int32, L)
    ones = jnp.full((L,), 1.0, jnp.float32)
    zeros = jnp.zeros((L,), jnp.float32)

    def zero_hist(h):
        def b(i, _):
            h[i, :] = zeros
            return 0
        lax.fori_loop(0, BINS + 1, b, 0)

    def hist_array(arr_ref, h):
        def consume(buf, carry):
            @plsc.parallel_loop(0, NVEC, unroll=8)
            def vec_body(i):
                x = buf[i // (COLS // L), pl.ds((i % (COLS // L)) * L, L)]
                scaled = (x - lo) / w * jnp.float32(BINS)
                idx = scaled.astype(jnp.int32)
                idxu = plsc.bitcast(idx, jnp.uint32)
                valid = (x >= lo) & (idxu <= jnp.uint32(BINS))
                plsc.addupdate_scatter(h, [idx, lane], ones, mask=valid)

            return carry

        _stream_chunks(arr_ref, rbase, (buf0, buf1), (sem0, sem1), consume, 0)

    zero_hist(hp)
    zero_hist(ht)
    hist_array(p_hbm, hp)
    hist_array(t_hbm, ht)
    pltpu.sync_copy(hp, out_hbm.at[_wid(), 0])
    pltpu.sync_copy(ht, out_hbm.at[_wid(), 1])


def kernel(prediction, target):
    p = prediction.reshape(ROWS, COLS)
    t = target.reshape(ROWS, COLS)
    mm = _minmax_kernel(p, t)
    minv = jnp.min(mm[:, 0, :])
    maxv = jnp.max(mm[:, 1, :])
    lo = minv + 0.1
    width = maxv - lo
    lo16 = jnp.full((L,), lo, jnp.float32)
    w16 = jnp.full((L,), width, jnp.float32)
    parts = _hist_kernel(p, t, lo16, w16)
    h = parts.sum(axis=(0, 3))
    h = h[:, :BINS].at[:, BINS - 1].add(h[:, BINS])
    return jnp.mean(jnp.abs(h[0] - h[1]))


# speedup vs baseline: 219.0968x; 1.0068x over previous
import functools

import jax
import jax.numpy as jnp
from jax import lax
from jax.experimental import pallas as pl
from jax.experimental.pallas import tpu as pltpu, tpu_sc as plsc

BINS = 100
NC = 2
NS = 16
NW = NC * NS
L = 16

N = 32 * 3 * 512 * 512
COLS = 512
ROWS = N // COLS
ROWS_PER_TILE = ROWS // NW
CROWS = 64
CHUNK = CROWS * COLS
NCHUNK = ROWS_PER_TILE // CROWS
NVEC = CHUNK // L

_mesh = plsc.VectorSubcoreMesh(core_axis_name="c", subcore_axis_name="s")
_params = pltpu.CompilerParams(needs_layout_passes=False,
                               use_tc_tiling_on_sc=True)


def _wid():
    return lax.axis_index("s") * NC + lax.axis_index("c")


def _stream_chunks(arr_ref, rbase, bufs, sems, consume, carry):
    for b in range(2):
        pltpu.async_copy(
            arr_ref.at[pl.ds(rbase + b * CROWS, CROWS), :], bufs[b], sems[b])

    def pair_body(g, carry):
        for b in range(2):
            c = 2 * g + b
            pltpu.make_async_copy(
                arr_ref.at[pl.ds(0, CROWS), :], bufs[b], sems[b]).wait()
            carry = consume(bufs[b], carry)

            @pl.when(c + 2 < NCHUNK)
            def _():
                pltpu.async_copy(
                    arr_ref.at[pl.ds(rbase + (c + 2) * CROWS, CROWS), :],
                    bufs[b], sems[b])
        return carry

    return lax.fori_loop(0, NCHUNK // 2, pair_body, carry)


@functools.partial(
    pl.kernel,
    out_type=jax.ShapeDtypeStruct((NW, 2, L), jnp.float32),
    mesh=_mesh,
    compiler_params=_params,
    scratch_types=[
        pltpu.VMEM((CROWS, COLS), jnp.float32),
        pltpu.VMEM((CROWS, COLS), jnp.float32),
        pltpu.VMEM((2, L), jnp.float32),
        pltpu.SemaphoreType.DMA,
        pltpu.SemaphoreType.DMA,
    ],
)
def _minmax_kernel(p_hbm, t_hbm, out_hbm, buf0, buf1, mm, sem0, sem1):
    rbase = _wid() * ROWS_PER_TILE

    def consume(buf, carry):
        @plsc.parallel_loop(0, NVEC, unroll=8, carry=carry)
        def vec_body(i, carry):
            vmin, vmax = carry
            x = buf[i // (COLS // L), pl.ds((i % (COLS // L)) * L, L)]
            return jnp.minimum(vmin, x), jnp.maximum(vmax, x)

        return vec_body

    def scan_array(arr_ref, carry):
        return _stream_chunks(
            arr_ref, rbase, (buf0, buf1), (sem0, sem1), consume, carry)

    init = (jnp.full((L,), jnp.inf, jnp.float32),
            jnp.full((L,), -jnp.inf, jnp.float32))
    vmin, vmax = scan_array(t_hbm, scan_array(p_hbm, init))
    mm[0, :] = vmin
    mm[1, :] = vmax
    pltpu.sync_copy(mm, out_hbm.at[_wid()])


@functools.partial(
    pl.kernel,
    out_type=jax.ShapeDtypeStruct((NW, 2, BINS + 1, L), jnp.float32),
    mesh=_mesh,
    compiler_params=_params,
    scratch_types=[
        pltpu.VMEM((CROWS, COLS), jnp.float32),
        pltpu.VMEM((CROWS, COLS), jnp.float32),
        pltpu.VMEM((BINS + 1, L), jnp.float32),
        pltpu.VMEM((BINS + 1, L), jnp.float32),
        pltpu.VMEM((NW, 2, L), jnp.float32),
        pltpu.SemaphoreType.DMA,
        pltpu.SemaphoreType.DMA,
    ],
)
def _hist_kernel(p_hbm, t_hbm, mm_hbm, out_hbm,
                 buf0, buf1, hp, ht, mmv, sem0, sem1):
    rbase = _wid() * ROWS_PER_TILE
    pltpu.sync_copy(mm_hbm, mmv)

    def mm_fold(i, carry):
        vmn, vmx = carry
        return (jnp.minimum(vmn, mmv[i, 0, :]), jnp.maximum(vmx, mmv[i, 1, :]))

    vmn, vmx = lax.fori_loop(
        0, NW, mm_fold,
        (jnp.full((L,), jnp.inf, jnp.float32),
         jnp.full((L,), -jnp.inf, jnp.float32)))
    minv = jnp.min(vmn)
    maxv = jnp.max(vmx)
    lo_s = minv + jnp.float32(0.1)
    lo = jnp.full((L,), lo_s, jnp.float32)
    w = jnp.full((L,), maxv - lo_s, jnp.float32)
    lane = lax.iota(jnp.int32, L)
    ones = jnp.full((L,), 1.0, jnp.float32)
    zeros = jnp.zeros((L,), jnp.float32)

    def zero_hist(h):
        def b(i, _):
            h[i, :] = zeros
            return 0
        lax.fori_loop(0, BINS + 1, b, 0)

    def hist_array(arr_ref, h):
        def consume(buf, carry):
            @plsc.parallel_loop(0, NVEC, unroll=8)
            def vec_body(i):
                x = buf[i // (COLS // L), pl.ds((i % (COLS // L)) * L, L)]
                scaled = (x - lo) / w * jnp.float32(BINS)
                idx = scaled.astype(jnp.int32)
                idxu = plsc.bitcast(idx, jnp.uint32)
                valid = (x >= lo) & (idxu <= jnp.uint32(BINS))
                plsc.addupdate_scatter(h, [idx, lane], ones, mask=valid)

            return carry

        _stream_chunks(arr_ref, rbase, (buf0, buf1), (sem0, sem1), consume, 0)

    zero_hist(hp)
    zero_hist(ht)
    hist_array(p_hbm, hp)
    hist_array(t_hbm, ht)
    pltpu.sync_copy(hp, out_hbm.at[_wid(), 0])
    pltpu.sync_copy(ht, out_hbm.at[_wid(), 1])


def kernel(prediction, target):
    p = prediction.reshape(ROWS, COLS)
    t = target.reshape(ROWS, COLS)
    mm = _minmax_kernel(p, t)
    parts = _hist_kernel(p, t, mm)
    h = parts.sum(axis=(0, 3))
    h = h[:, :BINS].at[:, BINS - 1].add(h[:, BINS])
    return jnp.mean(jnp.abs(h[0] - h[1]))
